# Initial kernel scaffold; baseline (speedup 1.0000x reference)
import jax
import jax.numpy as jnp
from jax.experimental import pallas as pl


def kernel(feat, Wg, bg, segment_ids):
    raise NotImplementedError("write your pallas kernel here")


# speedup vs baseline: 9.1399x; 9.1399x over previous
import functools

import jax
import jax.numpy as jnp
from jax import lax
from jax.experimental import pallas as pl
from jax.experimental.pallas import tpu as pltpu
from jax.experimental.pallas import tpu_sc as plsc

N_NODES = 100000
D = 128
S = 256
NC, NS, L = 2, 16, 16
NW = NC * NS
SEG_PER_W = S // NW
CH = 3136
CAND = 272
SENT = N_NODES
NBUF = 4

_mesh = plsc.VectorSubcoreMesh(
    core_axis_name="c", subcore_axis_name="s", num_cores=NC, num_subcores=NS)


def _iota():
    return lax.iota(jnp.int32, L)


def _lane_bcast(v, j):
    idx = jnp.full((L,), j, jnp.int32)
    return v.at[idx].get(mode="promise_in_bounds")


def _worker_id():
    return lax.axis_index("c") * NS + lax.axis_index("s")


def _scalar_at(ref, idx):
    base = (idx // L) * L
    blk = ref[pl.ds(base, L)]
    return _lane_bcast(blk, idx - base)[0]


@functools.partial(
    pl.kernel,
    out_type=jax.ShapeDtypeStruct((NW, CAND), jnp.int32),
    mesh=_mesh,
    compiler_params=pltpu.CompilerParams(needs_layout_passes=False),
    scratch_types=[
        pltpu.VMEM((CH,), jnp.int32),
        pltpu.VMEM((CAND,), jnp.int32),
        pltpu.VMEM((L,), jnp.int32),
    ],
)
def _starts_kernel(ids_hbm, cand_hbm, ids_v, cand_v, prev_v):
    w = _worker_id()
    base = jnp.minimum(w * CH, N_NODES - CH)
    for i in range(CAND // L):
        cand_v[pl.ds(i * L, L)] = jnp.full((L,), SENT, jnp.int32)
    pltpu.sync_copy(ids_hbm.at[pl.ds(base, CH)], ids_v)

    @pl.when(base > 0)
    def _():
        pltpu.sync_copy(ids_hbm.at[pl.ds(base - L, L)], prev_v)

    prev0 = jnp.where(base > 0,
                      _lane_bcast(prev_v[...], L - 1),
                      jnp.full((L,), -1, jnp.int32))
    iota = _iota()
    shift_idx = jnp.maximum(iota - 1, 0)

    def body(k, prev):
        ids = ids_v[pl.ds(k * L, L)]
        shifted = ids.at[shift_idx].get(mode="promise_in_bounds")
        prev_vec = jnp.where(iota == 0, prev, shifted)
        change = ids != prev_vec
        rows = base + k * L + iota
        plsc.store_scatter(cand_v, [ids], rows, mask=change)
        return _lane_bcast(ids, L - 1)

    lax.fori_loop(0, CH // L, body, prev0)
    pltpu.sync_copy(cand_v, cand_hbm.at[w])


@functools.partial(
    pl.kernel,
    out_type=jax.ShapeDtypeStruct((S, D), jnp.float32),
    mesh=_mesh,
    compiler_params=pltpu.CompilerParams(needs_layout_passes=False),
    scratch_types=[
        pltpu.VMEM((NW, CAND), jnp.int32),
        pltpu.VMEM((CAND,), jnp.int32),
        pltpu.VMEM((NBUF, L, D), jnp.float32),
        pltpu.VMEM((D,), jnp.float32),
        pltpu.VMEM((SEG_PER_W, D), jnp.float32),
        pltpu.SemaphoreType.DMA((NBUF,)),
    ],
)
def _main_kernel(feat_hbm, wg_hbm, cand_hbm, out_hbm,
                 cand_v, start_v, f_v, wg_v, stage_v, sems):
    w = _worker_id()
    pltpu.sync_copy(wg_hbm, wg_v)
    pltpu.sync_copy(cand_hbm, cand_v)

    for v in range(CAND // L):
        m = cand_v[0, pl.ds(v * L, L)]
        for t in range(1, NW):
            m = jnp.minimum(m, cand_v[t, pl.ds(v * L, L)])
        start_v[pl.ds(v * L, L)] = m

    carry = jnp.full((L,), SENT, jnp.int32)
    for v in range(CAND // L - 1, -1, -1):
        blk = start_v[pl.ds(v * L, L)]
        suf = -lax.rev(plsc.cummax(lax.rev(-blk, (0,))), (0,))
        res = jnp.minimum(suf, carry)
        start_v[pl.ds(v * L, L)] = res
        carry = _lane_bcast(res, 0)

    r_lo = _scalar_at(start_v, w * SEG_PER_W)
    a_lo = (r_lo // L) * L

    wgs = [wg_v[pl.ds(i * L, L)] for i in range(D // L)]
    fzero = jnp.zeros((L,), jnp.float32)
    fone = jnp.ones((L,), jnp.float32)

    def seg_body(s, _):
        sidx = w * SEG_PER_W + s
        aa = _scalar_at(start_v, sidx)
        bb = _scalar_at(start_v, sidx + 1)
        k0 = (aa - a_lo) // L
        k1 = (bb - a_lo + L - 1) // L
        cnt = k1 - k0
        n_outer = (cnt + NBUF - 1) // NBUF
        total = n_outer * NBUF

        def src_for(j):
            k_eff = k0 + jnp.minimum(j, cnt - 1)
            return feat_hbm.at[pl.ds(a_lo + k_eff * L, L)]

        for b in range(NBUF - 1):
            @pl.when(b < total)
            def _():
                pltpu.async_copy(src_for(b), f_v.at[b], sems.at[b])

        def outer(c, carry):
            shift, s_acc, accs = carry
            for b in range(NBUF):
                i = c * NBUF + b
                pj = i + NBUF - 1
                pb = (b + NBUF - 1) % NBUF

                @pl.when(pj < total)
                def _():
                    pltpu.async_copy(src_for(pj), f_v.at[pb], sems.at[pb])

                pltpu.make_async_copy(src_for(i), f_v.at[b], sems.at[b]).wait()
                row0 = a_lo + (k0 + i) * L
                use = i < cnt
                for j in range(L):
                    fs = [f_v[b, j, pl.ds(ii * L, L)] for ii in range(D // L)]
                    dot = fs[0] * wgs[0]
                    for ii in range(1, D // L):
                        dot = dot + fs[ii] * wgs[ii]
                    gsp = jnp.full((L,), jnp.sum(dot))
                    rowj = row0 + j
                    shift = jnp.where(rowj == aa, gsp, shift)
                    e = jnp.exp(gsp - shift)
                    em = jnp.where((rowj >= aa) & (rowj < bb) & use, e, fzero)
                    s_acc = s_acc + em
                    accs = tuple(a + em * f for a, f in zip(accs, fs))
            return shift, s_acc, accs

        init = (fzero, fzero, tuple(fzero for _ in range(D // L)))
        _, s_acc, accs = lax.fori_loop(0, n_outer, outer, init)
        rden = jnp.where(s_acc > 0, fone / s_acc, fzero)
        for ii in range(D // L):
            stage_v[s, pl.ds(ii * L, L)] = accs[ii] * rden
        return 0

    lax.fori_loop(0, SEG_PER_W, seg_body, 0)
    pltpu.sync_copy(stage_v, out_hbm.at[pl.ds(w * SEG_PER_W, SEG_PER_W)])


def kernel(feat, Wg, bg, segment_ids):
    del bg
    ids32 = segment_ids.astype(jnp.int32)
    wg = Wg.reshape(D).astype(jnp.float32)
    cands = _starts_kernel(ids32)
    return _main_kernel(feat, wg, cands)


# speedup vs baseline: 17.4893x; 1.9135x over previous
import functools

import jax
import jax.numpy as jnp
from jax import lax
from jax.experimental import pallas as pl
from jax.experimental.pallas import tpu as pltpu
from jax.experimental.pallas import tpu_sc as plsc

N_NODES = 100000
D = 128
S = 256
NC, NS, L = 2, 16, 16
NW = NC * NS
SEG_PER_W = S // NW
CH = 3136
CAND = 272
SENT = N_NODES
NBUF = 4

_mesh = plsc.VectorSubcoreMesh(
    core_axis_name="c", subcore_axis_name="s", num_cores=NC, num_subcores=NS)


def _iota():
    return lax.iota(jnp.int32, L)


def _lane_bcast(v, j):
    idx = jnp.full((L,), j, jnp.int32)
    return v.at[idx].get(mode="promise_in_bounds")


def _worker_id():
    return lax.axis_index("c") * NS + lax.axis_index("s")


def _scalar_at(ref, idx):
    base = (idx // L) * L
    blk = ref[pl.ds(base, L)]
    return _lane_bcast(blk, idx - base)[0]


@functools.partial(
    pl.kernel,
    out_type=jax.ShapeDtypeStruct((NW, CAND), jnp.int32),
    mesh=_mesh,
    compiler_params=pltpu.CompilerParams(needs_layout_passes=False),
    scratch_types=[
        pltpu.VMEM((CH,), jnp.int32),
        pltpu.VMEM((CAND,), jnp.int32),
        pltpu.VMEM((L,), jnp.int32),
    ],
)
def _starts_kernel(ids_hbm, cand_hbm, ids_v, cand_v, prev_v):
    w = _worker_id()
    base = jnp.minimum(w * CH, N_NODES - CH)
    for i in range(CAND // L):
        cand_v[pl.ds(i * L, L)] = jnp.full((L,), SENT, jnp.int32)
    pltpu.sync_copy(ids_hbm.at[pl.ds(base, CH)], ids_v)

    @pl.when(base > 0)
    def _():
        pltpu.sync_copy(ids_hbm.at[pl.ds(base - L, L)], prev_v)

    prev0 = jnp.where(base > 0,
                      _lane_bcast(prev_v[...], L - 1),
                      jnp.full((L,), -1, jnp.int32))
    iota = _iota()
    shift_idx = jnp.maximum(iota - 1, 0)

    def body(k, prev):
        ids = ids_v[pl.ds(k * L, L)]
        shifted = ids.at[shift_idx].get(mode="promise_in_bounds")
        prev_vec = jnp.where(iota == 0, prev, shifted)
        change = ids != prev_vec
        rows = base + k * L + iota
        plsc.store_scatter(cand_v, [ids], rows, mask=change)
        return _lane_bcast(ids, L - 1)

    lax.fori_loop(0, CH // L, body, prev0)
    pltpu.sync_copy(cand_v, cand_hbm.at[w])


@functools.partial(
    pl.kernel,
    out_type=jax.ShapeDtypeStruct((S, D), jnp.float32),
    mesh=_mesh,
    compiler_params=pltpu.CompilerParams(needs_layout_passes=False),
    scratch_types=[
        pltpu.VMEM((NW, CAND), jnp.int32),
        pltpu.VMEM((CAND,), jnp.int32),
        pltpu.VMEM((NBUF, L, D), jnp.float32),
        pltpu.VMEM((D,), jnp.float32),
        pltpu.VMEM((SEG_PER_W, D), jnp.float32),
        pltpu.VMEM((D,), jnp.float32),
        pltpu.SemaphoreType.DMA((NBUF,)),
    ],
)
def _main_kernel(feat_hbm, wg_hbm, cand_hbm, out_hbm,
                 cand_v, start_v, f_v, wg_v, stage_v, fr_v, sems):
    w = _worker_id()
    pltpu.sync_copy(wg_hbm, wg_v)
    pltpu.sync_copy(cand_hbm, cand_v)

    for v in range(CAND // L):
        m = cand_v[0, pl.ds(v * L, L)]
        for t in range(1, NW):
            m = jnp.minimum(m, cand_v[t, pl.ds(v * L, L)])
        start_v[pl.ds(v * L, L)] = m

    carry = jnp.full((L,), SENT, jnp.int32)
    for v in range(CAND // L - 1, -1, -1):
        blk = start_v[pl.ds(v * L, L)]
        suf = -lax.rev(plsc.cummax(lax.rev(-blk, (0,))), (0,))
        res = jnp.minimum(suf, carry)
        start_v[pl.ds(v * L, L)] = res
        carry = _lane_bcast(res, 0)

    r_lo = _scalar_at(start_v, w * SEG_PER_W)
    a_lo = (r_lo // L) * L

    wgs = [wg_v[pl.ds(i * L, L)] for i in range(D // L)]
    fzero = jnp.zeros((L,), jnp.float32)
    fone = jnp.ones((L,), jnp.float32)
    iota = _iota()

    def seg_body(s, _):
        sidx = w * SEG_PER_W + s
        aa = _scalar_at(start_v, sidx)
        bb = _scalar_at(start_v, sidx + 1)
        k0 = (aa - a_lo) // L
        k1 = (bb - a_lo + L - 1) // L
        cnt = k1 - k0
        n_outer = (cnt + NBUF - 1) // NBUF
        total = n_outer * NBUF

        pltpu.sync_copy(feat_hbm.at[jnp.minimum(aa, N_NODES - 1)], fr_v)
        dsh = fr_v[pl.ds(0, L)] * wgs[0]
        for ii in range(1, D // L):
            dsh = dsh + fr_v[pl.ds(ii * L, L)] * wgs[ii]
        shift = _lane_bcast(plsc.cumsum(dsh), L - 1)

        def src_for(j):
            k_eff = k0 + jnp.minimum(j, cnt - 1)
            return feat_hbm.at[pl.ds(a_lo + k_eff * L, L)]

        for b in range(NBUF - 1):
            @pl.when(b < total)
            def _():
                pltpu.async_copy(src_for(b), f_v.at[b], sems.at[b])

        def outer(c, carry):
            s_acc, accs = carry
            for b in range(NBUF):
                i = c * NBUF + b
                pj = i + NBUF - 1
                pb = (b + NBUF - 1) % NBUF

                @pl.when(pj < total)
                def _():
                    pltpu.async_copy(src_for(pj), f_v.at[pb], sems.at[pb])

                pltpu.make_async_copy(src_for(i), f_v.at[b], sems.at[b]).wait()
                row0 = a_lo + (k0 + i) * L
                use = i < cnt
                for j in range(L):
                    fs = [f_v[b, j, pl.ds(ii * L, L)] for ii in range(D // L)]
                    ms = [f * wg for f, wg in zip(fs, wgs)]
                    while len(ms) > 1:
                        ms = [ms[p] + ms[p + 1] for p in range(0, len(ms), 2)]
                    e = jnp.exp(_lane_bcast(plsc.cumsum(ms[0]), L - 1) - shift)
                    rowj = row0 + j
                    ok = (rowj >= aa) & (rowj < bb) & use
                    em = jnp.where(ok, e, fzero)
                    s_acc = s_acc + em
                    accs = tuple(a + em * f for a, f in zip(accs, fs))
            return s_acc, accs

        init = (fzero, tuple(fzero for _ in range(D // L)))
        s_acc, accs = lax.fori_loop(0, n_outer, outer, init)
        rden = jnp.where(s_acc > 0, fone / s_acc, fzero)
        for ii in range(D // L):
            stage_v[s, pl.ds(ii * L, L)] = accs[ii] * rden
        return 0

    lax.fori_loop(0, SEG_PER_W, seg_body, 0)
    pltpu.sync_copy(stage_v, out_hbm.at[pl.ds(w * SEG_PER_W, SEG_PER_W)])


def kernel(feat, Wg, bg, segment_ids):
    del bg
    ids32 = segment_ids.astype(jnp.int32)
    wg = Wg.reshape(D).astype(jnp.float32)
    cands = _starts_kernel(ids32)
    return _main_kernel(feat, wg, cands)
